# baseline (speedup 1.0000x reference)
import functools

import jax
import jax.numpy as jnp
import numpy as np
from jax import lax
from jax.experimental import pallas as pl
from jax.experimental.pallas import tpu as pltpu
from jax.experimental.pallas import tpu_sc as plsc

CH = 256
STEPS = 8192
NQ = 16384
NC = 2
NS = 16
L = 16
NW = NC * NS
QPW = NQ // NW
CHUNK = 64
NCHUNK = QPW // CHUNK
NBUF = 2


_mesh = plsc.VectorSubcoreMesh(core_axis_name="c", subcore_axis_name="s")


@functools.partial(
    pl.kernel,
    out_type=jax.ShapeDtypeStruct((NQ, CH), jnp.float32),
    mesh=_mesh,
    scratch_types=[
        pltpu.VMEM((QPW,), jnp.float32),
        pltpu.VMEM((QPW,), jnp.int32),
        pltpu.VMEM((QPW,), jnp.int32),
        pltpu.VMEM((QPW,), jnp.float32),
        pltpu.VMEM((CHUNK, CH // 2), jnp.int32),
        pltpu.VMEM((CHUNK, CH // 2), jnp.int32),
        pltpu.VMEM((CHUNK, CH // 2), jnp.int32),
        pltpu.VMEM((CHUNK, CH // 2), jnp.int32),
        pltpu.VMEM((CHUNK, CH), jnp.float32),
        pltpu.VMEM((CHUNK, CH), jnp.float32),
        pltpu.SemaphoreType.DMA,
        pltpu.SemaphoreType.DMA,
        pltpu.SemaphoreType.DMA,
        pltpu.SemaphoreType.DMA,
    ],
)
def _interp_kernel(t_hbm, control_hbm, out_hbm,
                   t_v, i0_v, i1_v, f_v,
                   r0a, r0b, r1a, r1b, oa, ob,
                   gsa, gsb, ssa, ssb):
    r0 = (r0a, r0b)
    r1 = (r1a, r1b)
    o = (oa, ob)
    gs = (gsa, gsb)
    ss = (ssa, ssb)

    wid = lax.axis_index("s") * NC + lax.axis_index("c")
    base = wid * QPW

    pltpu.sync_copy(t_hbm.at[pl.ds(base, QPW)], t_v)

    def precompute(i, carry):
        tv = t_v[pl.ds(i * L, L)]
        pos = tv * jnp.float32(STEPS - 1)
        i0 = pos.astype(jnp.int32)
        i0 = jnp.maximum(jnp.minimum(i0, STEPS - 2), 0)
        fr = pos - i0.astype(jnp.float32)
        i0_v[pl.ds(i * L, L)] = i0
        i1_v[pl.ds(i * L, L)] = i0 + 1
        f_v[pl.ds(i * L, L)] = fr
        return carry

    lax.fori_loop(0, QPW // L, precompute, 0)

    def fire_gathers(c, b):
        cb = c * CHUNK
        pltpu.async_copy(
            control_hbm.at[i0_v.at[pl.ds(cb, CHUNK)]], r0[b], gs[b])
        pltpu.async_copy(
            control_hbm.at[i1_v.at[pl.ds(cb, CHUNK)]], r1[b], gs[b])

    def wait_gathers(c, b):
        cb = c * CHUNK
        pltpu.make_async_copy(
            control_hbm.at[i0_v.at[pl.ds(cb, CHUNK)]], r0[b], gs[b]).wait()
        pltpu.make_async_copy(
            control_hbm.at[i1_v.at[pl.ds(cb, CHUNK)]], r1[b], gs[b]).wait()

    fire_gathers(0, 0)
    fire_gathers(1, 1)

    def outer(k, carry):
        for b in range(NBUF):
            c = NBUF * k + b
            cb = c * CHUNK
            wait_gathers(c, b)

            @pl.when(k > 0)
            def _():
                pltpu.make_async_copy(
                    o[b], out_hbm.at[pl.ds(base, CHUNK)], ss[b]).wait()

            def blend(qg, carry2, b=b, cb=cb):
                fr16 = f_v[pl.ds(cb + qg * L, L)]
                for j in range(L):
                    q = qg * L + j
                    fq = jnp.full((L,), fr16[j])
                    for g in range(CH // 32):
                        w0 = r0[b][q, pl.ds(g * L, L)]
                        w1 = r1[b][q, pl.ds(g * L, L)]
                        a0 = lax.bitcast_convert_type(w0 << 16, jnp.float32)
                        b0 = lax.bitcast_convert_type(w0 & jnp.int32(-65536), jnp.float32)
                        a1 = lax.bitcast_convert_type(w1 << 16, jnp.float32)
                        b1 = lax.bitcast_convert_type(w1 & jnp.int32(-65536), jnp.float32)
                        o[b][q, pl.ds(g * L, L)] = a0 + fq * (a1 - a0)
                        o[b][q, pl.ds(CH // 2 + g * L, L)] = b0 + fq * (b1 - b0)
                return carry2

            lax.fori_loop(0, CHUNK // L, blend, 0)

            pltpu.async_copy(o[b], out_hbm.at[pl.ds(base + cb, CHUNK)], ss[b])

            @pl.when(c + NBUF < NCHUNK)
            def _(c=c, b=b):
                fire_gathers(c + NBUF, b)
        return carry

    lax.fori_loop(0, NCHUNK // NBUF, outer, 0)

    pltpu.make_async_copy(o[0], out_hbm.at[pl.ds(base, CHUNK)], ss[0]).wait()
    pltpu.make_async_copy(o[1], out_hbm.at[pl.ds(base, CHUNK)], ss[1]).wait()


def kernel(t, control):
    cb = control.astype(jnp.bfloat16).reshape(STEPS, 2, CH // 2)
    packed = jax.lax.bitcast_convert_type(
        jnp.transpose(cb, (0, 2, 1)), jnp.int32)
    return _interp_kernel(t, packed)


# speedup vs baseline: 1.6358x; 1.6358x over previous
import functools

import jax
import jax.numpy as jnp
from jax import lax
from jax.experimental import pallas as pl
from jax.experimental.pallas import tpu as pltpu
from jax.experimental.pallas import tpu_sc as plsc

CH = 256
STEPS = 8192
NQ = 16384
NC = 2
NS = 16
L = 16
NW = NC * NS
QPW = NQ // NW
CHUNK = 64
NCHUNK = QPW // CHUNK
NBUF = 2

_mesh = plsc.VectorSubcoreMesh(core_axis_name="c", subcore_axis_name="s")


@functools.partial(
    pl.kernel,
    out_type=jax.ShapeDtypeStruct((NQ, CH), jnp.float32),
    mesh=_mesh,
    scratch_types=[
        pltpu.VMEM((QPW,), jnp.float32),
        pltpu.VMEM((QPW,), jnp.int32),
        pltpu.VMEM((QPW,), jnp.int32),
        pltpu.VMEM((QPW * L,), jnp.float32),
        pltpu.VMEM((CHUNK, CH // 2), jnp.int32),
        pltpu.VMEM((CHUNK, CH // 2), jnp.int32),
        pltpu.VMEM((CHUNK, CH // 2), jnp.int32),
        pltpu.VMEM((CHUNK, CH // 2), jnp.int32),
        pltpu.VMEM((CHUNK, CH), jnp.float32),
        pltpu.VMEM((CHUNK, CH), jnp.float32),
        pltpu.SemaphoreType.DMA,
        pltpu.SemaphoreType.DMA,
        pltpu.SemaphoreType.DMA,
        pltpu.SemaphoreType.DMA,
    ],
)
def _interp_kernel(t_hbm, control_hbm, out_hbm,
                   t_v, i0_v, i1_v, f_b,
                   r0a, r0b, r1a, r1b, oa, ob,
                   gsa, gsb, ssa, ssb):
    r0 = (r0a, r0b)
    r1 = (r1a, r1b)
    o = (oa, ob)
    gs = (gsa, gsb)
    ss = (ssa, ssb)

    wid = lax.axis_index("s") * NC + lax.axis_index("c")
    base = wid * QPW

    pltpu.sync_copy(t_hbm.at[pl.ds(base, QPW)], t_v)

    @plsc.parallel_loop(0, QPW // L)
    def precompute(i):
        tv = t_v[pl.ds(i * L, L)]
        pos = tv * jnp.float32(STEPS - 1)
        i0 = pos.astype(jnp.int32)
        i0 = jnp.maximum(jnp.minimum(i0, STEPS - 2), 0)
        fr = pos - i0.astype(jnp.float32)
        i0_v[pl.ds(i * L, L)] = i0
        i1_v[pl.ds(i * L, L)] = i0 + 1
        for j in range(L):
            f_b[pl.ds((i * L + j) * L, L)] = jnp.full((L,), fr[j])

    def fire_gathers(c, b):
        cb = c * CHUNK
        pltpu.async_copy(
            control_hbm.at[i0_v.at[pl.ds(cb, CHUNK)]], r0[b], gs[b])
        pltpu.async_copy(
            control_hbm.at[i1_v.at[pl.ds(cb, CHUNK)]], r1[b], gs[b])

    def wait_gathers(c, b):
        cb = c * CHUNK
        pltpu.make_async_copy(
            control_hbm.at[i0_v.at[pl.ds(cb, CHUNK)]], r0[b], gs[b]).wait()
        pltpu.make_async_copy(
            control_hbm.at[i1_v.at[pl.ds(cb, CHUNK)]], r1[b], gs[b]).wait()

    fire_gathers(0, 0)
    fire_gathers(1, 1)

    def outer(k, carry):
        for b in range(NBUF):
            c = NBUF * k + b
            cb = c * CHUNK
            wait_gathers(c, b)

            @pl.when(k > 0)
            def _():
                pltpu.make_async_copy(
                    o[b], out_hbm.at[pl.ds(base, CHUNK)], ss[b]).wait()

            @plsc.parallel_loop(0, CHUNK, unroll=2)
            def blend(q, b=b, cb=cb):
                fq = f_b[pl.ds((cb + q) * L, L)]
                w0s = [r0[b][q, pl.ds(g * L, L)] for g in range(CH // 32)]
                w1s = [r1[b][q, pl.ds(g * L, L)] for g in range(CH // 32)]
                for g in range(CH // 32):
                    w0, w1 = w0s[g], w1s[g]
                    a0 = lax.bitcast_convert_type(w0 << 16, jnp.float32)
                    b0 = lax.bitcast_convert_type(
                        w0 & jnp.int32(-65536), jnp.float32)
                    a1 = lax.bitcast_convert_type(w1 << 16, jnp.float32)
                    b1 = lax.bitcast_convert_type(
                        w1 & jnp.int32(-65536), jnp.float32)
                    o[b][q, pl.ds(g * L, L)] = a0 + fq * (a1 - a0)
                    o[b][q, pl.ds(CH // 2 + g * L, L)] = b0 + fq * (b1 - b0)

            pltpu.async_copy(o[b], out_hbm.at[pl.ds(base + cb, CHUNK)], ss[b])

            @pl.when(c + NBUF < NCHUNK)
            def _(c=c, b=b):
                fire_gathers(c + NBUF, b)
        return carry

    lax.fori_loop(0, NCHUNK // NBUF, outer, 0)

    pltpu.make_async_copy(o[0], out_hbm.at[pl.ds(base, CHUNK)], ss[0]).wait()
    pltpu.make_async_copy(o[1], out_hbm.at[pl.ds(base, CHUNK)], ss[1]).wait()


def kernel(t, control):
    cb = control.astype(jnp.bfloat16).reshape(STEPS, 2, CH // 2)
    packed = jax.lax.bitcast_convert_type(
        jnp.transpose(cb, (0, 2, 1)), jnp.int32)
    return _interp_kernel(t, packed)


# speedup vs baseline: 3.9713x; 2.4277x over previous
import functools

import jax
import jax.numpy as jnp
from jax import lax
from jax.experimental import pallas as pl
from jax.experimental.pallas import tpu as pltpu
from jax.experimental.pallas import tpu_sc as plsc

CH = 256
STEPS = 8192
NQ = 16384
NC = 2
NS = 16
L = 16
NW = NC * NS
QPW = NQ // NW
CHUNK = 64
NCHUNK = QPW // CHUNK
NBUF = 2

_mesh = plsc.VectorSubcoreMesh(core_axis_name="c", subcore_axis_name="s")


@functools.partial(
    pl.kernel,
    out_type=jax.ShapeDtypeStruct((NQ, CH), jnp.float32),
    mesh=_mesh,
    scratch_types=[
        pltpu.VMEM((QPW,), jnp.float32),
        pltpu.VMEM((QPW,), jnp.int32),
        pltpu.VMEM((QPW,), jnp.int32),
        pltpu.VMEM((QPW * L,), jnp.float32),
        pltpu.VMEM((CHUNK, CH // 2), jnp.int32),
        pltpu.VMEM((CHUNK, CH // 2), jnp.int32),
        pltpu.VMEM((CHUNK, CH // 2), jnp.int32),
        pltpu.VMEM((CHUNK, CH // 2), jnp.int32),
        pltpu.VMEM((CHUNK, CH), jnp.float32),
        pltpu.VMEM((CHUNK, CH), jnp.float32),
        pltpu.SemaphoreType.DMA,
        pltpu.SemaphoreType.DMA,
        pltpu.SemaphoreType.DMA,
        pltpu.SemaphoreType.DMA,
    ],
)
def _interp_kernel(t_hbm, control_hbm, out_hbm,
                   t_v, i0_v, i1_v, f_b,
                   r0a, r0b, r1a, r1b, oa, ob,
                   gsa, gsb, ssa, ssb):
    r0 = (r0a, r0b)
    r1 = (r1a, r1b)
    o = (oa, ob)
    gs = (gsa, gsb)
    ss = (ssa, ssb)

    wid = lax.axis_index("s") * NC + lax.axis_index("c")
    base = wid * QPW

    pltpu.sync_copy(t_hbm.at[pl.ds(base, QPW)], t_v)

    @plsc.parallel_loop(0, QPW // L)
    def precompute(i):
        tv = t_v[pl.ds(i * L, L)]
        pos = tv * jnp.float32(STEPS - 1)
        i0 = pos.astype(jnp.int32)
        i0 = jnp.maximum(jnp.minimum(i0, STEPS - 2), 0)
        fr = pos - i0.astype(jnp.float32)
        i0_v[pl.ds(i * L, L)] = i0
        i1_v[pl.ds(i * L, L)] = i0 + 1
        for j in range(L):
            f_b[pl.ds((i * L + j) * L, L)] = jnp.full((L,), fr[j])

    def fire_gathers(c, b):
        cb = c * CHUNK
        pltpu.async_copy(
            control_hbm.at[i0_v.at[pl.ds(cb, CHUNK)]], r0[b], gs[b])
        pltpu.async_copy(
            control_hbm.at[i1_v.at[pl.ds(cb, CHUNK)]], r1[b], gs[b])

    def wait_gathers(c, b):
        cb = c * CHUNK
        pltpu.make_async_copy(
            control_hbm.at[i0_v.at[pl.ds(cb, CHUNK)]], r0[b], gs[b]).wait()
        pltpu.make_async_copy(
            control_hbm.at[i1_v.at[pl.ds(cb, CHUNK)]], r1[b], gs[b]).wait()

    fire_gathers(0, 0)
    fire_gathers(1, 1)

    def outer(k, carry):
        for b in range(NBUF):
            c = NBUF * k + b
            cb = c * CHUNK
            wait_gathers(c, b)

            @pl.when(k > 0)
            def _():
                pltpu.make_async_copy(
                    o[b], out_hbm.at[pl.ds(base, CHUNK)], ss[b]).wait()

            @plsc.parallel_loop(0, CHUNK, unroll=2)
            def blend(q, b=b, cb=cb):
                fq = f_b[pl.ds((cb + q) * L, L)]
                w0s = [r0[b][q, pl.ds(g * L, L)] for g in range(CH // 32)]
                w1s = [r1[b][q, pl.ds(g * L, L)] for g in range(CH // 32)]
                for g in range(CH // 32):
                    w0, w1 = w0s[g], w1s[g]
                    a0 = lax.bitcast_convert_type(w0 << 16, jnp.float32)
                    b0 = lax.bitcast_convert_type(
                        w0 & jnp.int32(-65536), jnp.float32)
                    a1 = lax.bitcast_convert_type(w1 << 16, jnp.float32)
                    b1 = lax.bitcast_convert_type(
                        w1 & jnp.int32(-65536), jnp.float32)
                    o[b][q, pl.ds(g * L, L)] = a0 + fq * (a1 - a0)
                    o[b][q, pl.ds(CH // 2 + g * L, L)] = b0 + fq * (b1 - b0)

            pltpu.async_copy(o[b], out_hbm.at[pl.ds(base + cb, CHUNK)], ss[b])

            @pl.when(c + NBUF < NCHUNK)
            def _(c=c, b=b):
                fire_gathers(c + NBUF, b)
        return carry

    lax.fori_loop(0, NCHUNK // NBUF, outer, 0)

    pltpu.make_async_copy(o[0], out_hbm.at[pl.ds(base, CHUNK)], ss[0]).wait()
    pltpu.make_async_copy(o[1], out_hbm.at[pl.ds(base, CHUNK)], ss[1]).wait()


def kernel(t, control):
    cbf = control.astype(jnp.bfloat16)
    lo = jax.lax.bitcast_convert_type(
        cbf[:, :CH // 2], jnp.uint16).astype(jnp.uint32)
    hi = jax.lax.bitcast_convert_type(
        cbf[:, CH // 2:], jnp.uint16).astype(jnp.uint32)
    packed = jax.lax.bitcast_convert_type(lo | (hi << 16), jnp.int32)
    return _interp_kernel(t, packed)
